# baseline (speedup 1.0000x reference)
import jax
import jax.numpy as jnp
from jax import lax
from jax.experimental import pallas as pl
from jax.experimental.pallas import tpu as pltpu
from jax.experimental.pallas import tpu_sc as plsc

_B, _N = 1024, 8192
_L = 16
_NW = 32
_ROWS_PER_W = _B // _NW
_NCHUNKS = _N // _L
_CPB = 4
_NT = _N // 128
_PPB = 128 // _L


def _hsum(vec, scratch):
    iota = lax.iota(jnp.int32, _L)
    for sh in (1, 2, 4, 8):
        scratch[...] = vec
        vec = vec + plsc.load_gather(scratch, [iota ^ sh])
    return vec


def _sc_body(x_hbm, out_hbm, r0, r1, r2, r3, ob0, ob1,
             si0, si1, si2, si3, so0, so1, hs):
    wid = lax.axis_index("s") * 2 + lax.axis_index("c")
    base = wid * _ROWS_PER_W
    ones = jnp.ones((_L,), jnp.float32)
    zeros = jnp.zeros((_L,), jnp.float32)
    two_n = jnp.float32(2 * _N)
    rows = (r0, r1, r2, r3)
    isems = (si0, si1, si2, si3)
    obufs = (ob0, ob1)
    osems = (so0, so1)

    def in_copy(r, b):
        return pltpu.make_async_copy(x_hbm.at[r], rows[b], isems[b])

    def out_copy(r, b):
        return pltpu.make_async_copy(obufs[b], out_hbm.at[r], osems[b])

    def decide(s, a):
        return jnp.where(a - s < two_n,
                         jnp.where(s + 3.0 * a > two_n, ones, zeros),
                         ones)

    def reduce_only(buf):
        @plsc.parallel_loop(0, _NCHUNKS, step=_CPB, unroll=2,
                            carry=(zeros,) * (2 * _CPB))
        def acc(j, carry):
            carry = list(carry)
            for c in range(_CPB):
                x = buf[pl.ds((j + c) * _L, _L)]
                carry[2 * c] = carry[2 * c] + x
                carry[2 * c + 1] = carry[2 * c + 1] + jnp.abs(x)
            return tuple(carry)

        s = _hsum(acc[0] + acc[2] + acc[4] + acc[6], hs)
        a = _hsum(acc[1] + acc[3] + acc[5] + acc[7], hs)
        return decide(s, a)

    def fill_only(buf, fill_v, ob):
        @plsc.parallel_loop(0, _NT, unroll=2)
        def fill(t):
            for pos in range(_PPB):
                x = buf[pl.ds(t * 128 + pos * _L, _L)]
                v = jnp.where(x == -1.0, fill_v, x)
                ob[2 * t, pl.ds(pos * _L, _L)] = ones - v
                ob[2 * t + 1, pl.ds(pos * _L, _L)] = v

    def merged(cur, nxt, ob, fill_v):
        @plsc.parallel_loop(0, _NT, unroll=2, carry=(zeros,) * 8)
        def acc(t, carry):
            carry = list(carry)
            for pos in range(_PPB):
                x = cur[pl.ds(t * 128 + pos * _L, _L)]
                v = jnp.where(x == -1.0, fill_v, x)
                ob[2 * t, pl.ds(pos * _L, _L)] = ones - v
                ob[2 * t + 1, pl.ds(pos * _L, _L)] = v
                y = nxt[pl.ds(t * 128 + pos * _L, _L)]
                p = pos % 4
                carry[2 * p] = carry[2 * p] + y
                carry[2 * p + 1] = carry[2 * p + 1] + jnp.abs(y)
            return tuple(carry)

        s = _hsum(acc[0] + acc[2] + acc[4] + acc[6], hs)
        a = _hsum(acc[1] + acc[3] + acc[5] + acc[7], hs)
        return decide(s, a)

    in_copy(base, 0).start()
    in_copy(base + 1, 1).start()
    in_copy(base, 0).wait()
    f_init = reduce_only(rows[0])

    @pl.loop(0, _ROWS_PER_W // 4, init_carry=f_init)
    def f_carry(g, f):
        last_g = _ROWS_PER_W // 4 - 1
        for k in range(4):
            r = base + 4 * g + k
            if k < 2:
                in_copy(r + 2, (k + 2) % 4).start()
                in_copy(r + 1, (k + 1) % 4).wait()
            else:
                @pl.when(g < last_g)
                def _():
                    in_copy(r + 2, (k + 2) % 4).start()
                if k == 2:
                    in_copy(r + 1, (k + 1) % 4).wait()
                else:
                    @pl.when(g < last_g)
                    def _():
                        in_copy(r + 1, (k + 1) % 4).wait()
            if k < 2:
                @pl.when(g > 0)
                def _():
                    out_copy(r - 2, k % 2).wait()
            else:
                out_copy(r - 2, k % 2).wait()
            f = merged(rows[k], rows[(k + 1) % 4], obufs[k % 2], f)
            out_copy(r, k % 2).start()
        return f

    out_copy(base + _ROWS_PER_W - 2, 0).wait()
    out_copy(base + _ROWS_PER_W - 1, 1).wait()


def kernel(X):
    mesh = plsc.VectorSubcoreMesh(core_axis_name="c", subcore_axis_name="s")
    f = pl.kernel(
        _sc_body,
        mesh=mesh,
        out_type=jax.ShapeDtypeStruct((_B, 2 * _NT, 128), jnp.float32),
        scratch_types=[
            pltpu.VMEM((_N,), jnp.float32),
            pltpu.VMEM((_N,), jnp.float32),
            pltpu.VMEM((_N,), jnp.float32),
            pltpu.VMEM((_N,), jnp.float32),
            pltpu.VMEM((2 * _NT, 128), jnp.float32),
            pltpu.VMEM((2 * _NT, 128), jnp.float32),
            pltpu.SemaphoreType.DMA,
            pltpu.SemaphoreType.DMA,
            pltpu.SemaphoreType.DMA,
            pltpu.SemaphoreType.DMA,
            pltpu.SemaphoreType.DMA,
            pltpu.SemaphoreType.DMA,
            pltpu.VMEM((_L,), jnp.float32),
        ],
        compiler_params=pltpu.CompilerParams(needs_layout_passes=False),
    )
    o = f(X)
    return (o.reshape(_B, _NT, 2, 128)
             .transpose(0, 1, 3, 2)
             .reshape(_B, _N, 2))


# speedup vs baseline: 1.0122x; 1.0122x over previous
import jax
import jax.numpy as jnp
from jax import lax
from jax.experimental import pallas as pl
from jax.experimental.pallas import tpu as pltpu
from jax.experimental.pallas import tpu_sc as plsc

_B, _N = 1024, 8192
_L = 16
_NW = 32
_ROWS_PER_W = _B // _NW
_NCHUNKS = _N // _L
_CPB = 4
_NT = _N // 128
_PPB = 128 // _L


def _hsum(vec, scratch):
    iota = lax.iota(jnp.int32, _L)
    for sh in (1, 2, 4, 8):
        scratch[...] = vec
        vec = vec + plsc.load_gather(scratch, [iota ^ sh])
    return vec


def _sc_body(x_hbm, out_hbm, r0, r1, r2, r3, ob0, ob1,
             si0, si1, si2, si3, so0, so1, hs):
    wid = lax.axis_index("s") * 2 + lax.axis_index("c")
    base = wid * _ROWS_PER_W
    ones = jnp.ones((_L,), jnp.float32)
    zeros = jnp.zeros((_L,), jnp.float32)
    two_n = jnp.float32(2 * _N)
    rows = (r0, r1, r2, r3)
    isems = (si0, si1, si2, si3)
    obufs = (ob0, ob1)
    osems = (so0, so1)

    def in_copy(r, b):
        return pltpu.make_async_copy(x_hbm.at[r], rows[b], isems[b])

    def out_copy(r, b):
        return pltpu.make_async_copy(obufs[b], out_hbm.at[r], osems[b])

    def decide(s, a):
        return jnp.where(a - s < two_n,
                         jnp.where(s + 3.0 * a > two_n, ones, zeros),
                         ones)

    def reduce_only(buf):
        @plsc.parallel_loop(0, _NCHUNKS, step=_CPB, unroll=2,
                            carry=(zeros,) * (2 * _CPB))
        def acc(j, carry):
            carry = list(carry)
            for c in range(_CPB):
                x = buf[pl.ds((j + c) * _L, _L)]
                carry[2 * c] = carry[2 * c] + x
                carry[2 * c + 1] = carry[2 * c + 1] + jnp.abs(x)
            return tuple(carry)

        s = _hsum(acc[0] + acc[2] + acc[4] + acc[6], hs)
        a = _hsum(acc[1] + acc[3] + acc[5] + acc[7], hs)
        return decide(s, a)

    def merged(cur, nxt, ob, fill_v):
        @plsc.parallel_loop(0, _NT, unroll=1, carry=(zeros,) * 8)
        def acc(t, carry):
            carry = list(carry)
            for pos in range(_PPB):
                x = cur[pl.ds(t * 128 + pos * _L, _L)]
                v = jnp.where(x == -1.0, fill_v, x)
                ob[2 * t, pl.ds(pos * _L, _L)] = ones - v
                ob[2 * t + 1, pl.ds(pos * _L, _L)] = v
                y = nxt[pl.ds(t * 128 + pos * _L, _L)]
                p = pos % 4
                carry[2 * p] = carry[2 * p] + y
                carry[2 * p + 1] = carry[2 * p + 1] + jnp.abs(y)
            return tuple(carry)

        s = _hsum(acc[0] + acc[2] + acc[4] + acc[6], hs)
        a = _hsum(acc[1] + acc[3] + acc[5] + acc[7], hs)
        return decide(s, a)

    in_copy(base, 0).start()
    in_copy(base + 1, 1).start()
    in_copy(base, 0).wait()
    f_init = reduce_only(rows[0])

    @pl.loop(0, _ROWS_PER_W // 4, init_carry=f_init)
    def f_carry(g, f):
        last_g = _ROWS_PER_W // 4 - 1
        for k in range(4):
            r = base + 4 * g + k
            if k < 2:
                in_copy(r + 2, (k + 2) % 4).start()
                in_copy(r + 1, (k + 1) % 4).wait()
            else:
                @pl.when(g < last_g)
                def _():
                    in_copy(r + 2, (k + 2) % 4).start()
                if k == 2:
                    in_copy(r + 1, (k + 1) % 4).wait()
                else:
                    @pl.when(g < last_g)
                    def _():
                        in_copy(r + 1, (k + 1) % 4).wait()
            if k < 2:
                @pl.when(g > 0)
                def _():
                    out_copy(r - 2, k % 2).wait()
            else:
                out_copy(r - 2, k % 2).wait()
            f = merged(rows[k], rows[(k + 1) % 4], obufs[k % 2], f)
            out_copy(r, k % 2).start()
        return f

    out_copy(base + _ROWS_PER_W - 2, 0).wait()
    out_copy(base + _ROWS_PER_W - 1, 1).wait()


def kernel(X):
    mesh = plsc.VectorSubcoreMesh(core_axis_name="c", subcore_axis_name="s")
    f = pl.kernel(
        _sc_body,
        mesh=mesh,
        out_type=jax.ShapeDtypeStruct((_B, 2 * _NT, 128), jnp.float32),
        scratch_types=[
            pltpu.VMEM((_N,), jnp.float32),
            pltpu.VMEM((_N,), jnp.float32),
            pltpu.VMEM((_N,), jnp.float32),
            pltpu.VMEM((_N,), jnp.float32),
            pltpu.VMEM((2 * _NT, 128), jnp.float32),
            pltpu.VMEM((2 * _NT, 128), jnp.float32),
            pltpu.SemaphoreType.DMA,
            pltpu.SemaphoreType.DMA,
            pltpu.SemaphoreType.DMA,
            pltpu.SemaphoreType.DMA,
            pltpu.SemaphoreType.DMA,
            pltpu.SemaphoreType.DMA,
            pltpu.VMEM((_L,), jnp.float32),
        ],
        compiler_params=pltpu.CompilerParams(needs_layout_passes=False),
    )
    o = f(X)
    return (o.reshape(_B, _NT, 2, 128)
             .transpose(0, 1, 3, 2)
             .reshape(_B, _N, 2))
